# Initial kernel scaffold; baseline (speedup 1.0000x reference)
import jax
import jax.numpy as jnp
from jax.experimental import pallas as pl


def kernel(input, pos, token_table, pos_table, gamma, beta, W, b):
    raise NotImplementedError("write your pallas kernel here")


# speedup vs baseline: 2.9964x; 2.9964x over previous
import jax
import jax.numpy as jnp
from jax import lax
from jax.experimental import pallas as pl
from jax.experimental.pallas import tpu as pltpu
from jax.experimental.pallas import tpu_sc as plsc

B = 4096
L = 200
H = 128
H_ATTN = 128
MAX_LEN = 200
POS_PAD = 256
N = B * L
EPS = 1e-5

GATHER_WINDOW = 128
TC_BLOCK = 1024


def _sc_gather(token_table, ids):
    mesh = plsc.VectorSubcoreMesh(core_axis_name="core", subcore_axis_name="subcore")

    @pl.kernel(out_type=jax.ShapeDtypeStruct((N, H), jnp.float32), mesh=mesh)
    def gather_kernel(tok_hbm, i_hbm, o_hbm):
        def body(i_vmem, o_vmem):
            pltpu.sync_copy(tok_hbm.at[i_vmem.at[0]], o_vmem)

        pltpu.emit_pipeline(
            body,
            grid=(N // GATHER_WINDOW,),
            in_specs=[pl.BlockSpec((1, GATHER_WINDOW), lambda i: (0, i))],
            out_specs=[pl.BlockSpec((GATHER_WINDOW, H), lambda i: (i, 0))],
            core_axis_name=("core", "subcore"),
            dimension_semantics=(pltpu.PARALLEL,),
        )(i_hbm, o_hbm)

    return gather_kernel(token_table, ids)


def _tc_body(rep_ref, pos_ref, ptab_ref, gamma_ref, beta_ref, wt_ref, b_ref, o_ref):
    rep = rep_ref[...]
    p = pos_ref[...]
    cols = lax.broadcasted_iota(jnp.int32, (TC_BLOCK, POS_PAD), 1)
    onehot = (p == cols).astype(jnp.float32)
    pos_e = jnp.dot(onehot, ptab_ref[...], preferred_element_type=jnp.float32)
    x = rep + pos_e
    mean = jnp.mean(x, axis=1, keepdims=True)
    xc = x - mean
    var = jnp.mean(xc * xc, axis=1, keepdims=True)
    xn = xc * lax.rsqrt(var + EPS)
    y = xn * gamma_ref[...] + beta_ref[...]
    o_ref[...] = jnp.dot(y, wt_ref[...], preferred_element_type=jnp.float32) + b_ref[...]


def _tc_ln_proj(rep, pos2d, ptab, gamma2d, beta2d, wt, b2d):
    return pl.pallas_call(
        _tc_body,
        grid=(N // TC_BLOCK,),
        in_specs=[
            pl.BlockSpec((TC_BLOCK, H), lambda i: (i, 0)),
            pl.BlockSpec((TC_BLOCK, 1), lambda i: (i, 0)),
            pl.BlockSpec((POS_PAD, H), lambda i: (0, 0)),
            pl.BlockSpec((1, H), lambda i: (0, 0)),
            pl.BlockSpec((1, H), lambda i: (0, 0)),
            pl.BlockSpec((H, H_ATTN), lambda i: (0, 0)),
            pl.BlockSpec((1, H_ATTN), lambda i: (0, 0)),
        ],
        out_specs=pl.BlockSpec((TC_BLOCK, H_ATTN), lambda i: (i, 0)),
        out_shape=jax.ShapeDtypeStruct((N, H_ATTN), jnp.float32),
    )(rep, pos2d, ptab, gamma2d, beta2d, wt, b2d)


def kernel(input, pos, token_table, pos_table, gamma, beta, W, b):
    ids = input.reshape(1, N).astype(jnp.int32)
    rep = _sc_gather(token_table, ids)
    pos2d = pos.reshape(N, 1).astype(jnp.int32)
    ptab = jnp.zeros((POS_PAD, H), jnp.float32).at[:MAX_LEN].set(pos_table)
    out2d = _tc_ln_proj(rep, pos2d, ptab, gamma.reshape(1, H),
                        beta.reshape(1, H), W.T, b.reshape(1, H_ATTN))
    return out2d.reshape(B, L, H_ATTN)
